# Initial kernel scaffold; baseline (speedup 1.0000x reference)
import jax
import jax.numpy as jnp
from jax.experimental import pallas as pl


def kernel(x, edge_index, params):
    raise NotImplementedError("write your pallas kernel here")


# speedup vs baseline: 6.6043x; 6.6043x over previous
import functools

import jax
import jax.numpy as jnp
from jax import lax
from jax.experimental import pallas as pl
from jax.experimental.pallas import tpu as pltpu
from jax.experimental.pallas import tpu_sc as plsc

_F32 = jnp.float32
_BLK = 2000


def _full_spec(shape):
    nd = len(shape)
    return pl.BlockSpec(shape, lambda i, _nd=nd: (0,) * _nd)


def _row_spec(f, blk=_BLK):
    return pl.BlockSpec((blk, f), lambda i: (i, 0))


def _agg_spec(f, blk=_BLK):
    return pl.BlockSpec((2, blk, f), lambda i: (0, i, 0))


def _mm(v, w_ref, b_ref):
    return jnp.dot(v, w_ref[...], preferred_element_type=_F32) + b_ref[...]


def _pa_body(x, w11, b11, w12, b12, w13, b13, p):
    xb = x[...]
    a = jnp.maximum(_mm(xb, w11, b11), 0.0)
    m = _mm(xb, w12, b12) * _mm(xb, w13, b13)
    p[...] = a + jnp.maximum(m, 0.0)


def _pbc_body(p_in, agg, wc, bc, wg1, bg1, wg2, bg2, w1, b1, w2, b2, w3, b3,
              p_out, u_out):
    a = agg[...]
    aggf = jnp.concatenate([a[0], a[1]], axis=-1)
    h = p_in[...] + jnp.maximum(_mm(aggf, wc, bc), 0.0)
    u = jax.nn.sigmoid(_mm(h, wg1, bg1)) * _mm(h, wg2, bg2)
    u_out[...] = u
    p_out[...] = (jnp.maximum(_mm(u, w1, b1), 0.0)
                  + jnp.maximum(_mm(u, w2, b2) * _mm(u, w3, b3), 0.0))


def _pd_body(p_in, agg, wc, bc, wf, bf, out):
    a = agg[...]
    aggf = a[0] + a[1]
    h = p_in[...] + jnp.maximum(_mm(aggf, wc, bc), 0.0)
    out[...] = _mm(h, wf, bf)


def _pass_a(x, prm, n):
    names = ["fc11_W", "fc11_b", "fc12_W", "fc12_b", "fc13_W", "fc13_b"]
    args = [prm[k] for k in names]
    return pl.pallas_call(
        _pa_body,
        grid=(n // _BLK,),
        in_specs=[_row_spec(64)] + [_full_spec(a.shape) for a in args],
        out_specs=_row_spec(64),
        out_shape=jax.ShapeDtypeStruct((n, 64), _F32),
    )(x, *args)


def _pass_bc(p_in, agg, prm, n, stage):
    if stage == 2:
        fi, fo = 64, 32
        names = ["conv11_W", "conv11_b", "gate1_fc1_W", "gate1_fc1_b",
                 "gate1_fc2_W", "gate1_fc2_b", "fc21_W", "fc21_b",
                 "fc22_W", "fc22_b", "fc23_W", "fc23_b"]
    else:
        fi, fo = 32, 16
        names = ["conv21_W", "conv21_b", "gate2_fc1_W", "gate2_fc1_b",
                 "gate2_fc2_W", "gate2_fc2_b", "fc31_W", "fc31_b",
                 "fc32_W", "fc32_b", "fc33_W", "fc33_b"]
    args = [prm[k] for k in names]
    return pl.pallas_call(
        _pbc_body,
        grid=(n // _BLK,),
        in_specs=([_row_spec(fi), _agg_spec(fi // 2)]
                  + [_full_spec(a.shape) for a in args]),
        out_specs=[_row_spec(fo), _row_spec(fo)],
        out_shape=[jax.ShapeDtypeStruct((n, fo), _F32)] * 2,
    )(p_in, agg, *args)


def _pass_d(p_in, agg, prm, n):
    names = ["conv31_W", "conv31_b", "fc2_W", "fc2_b"]
    args = [prm[k] for k in names]
    return pl.pallas_call(
        _pd_body,
        grid=(n // _BLK,),
        in_specs=([_row_spec(16), _agg_spec(16)]
                  + [_full_spec(a.shape) for a in args]),
        out_specs=_row_spec(16),
        out_shape=jax.ShapeDtypeStruct((n, 16), _F32),
    )(p_in, agg, *args)


_CH = 128
_NSUP = 8
_ZR = 112


def _sc_segsum(gflat, src2, dst2, n_pad, dh, column_split):
    e_chunks, ch = src2.shape
    assert ch == _CH
    nc, ns = 2, 16
    n_ch_w = e_chunks // ns if column_split else e_chunks // (ns * nc)
    n_sup = n_ch_w // _NSUP
    rows_w = n_pad // ns
    nz = rows_w // _ZR
    assert n_ch_w % 8 == 0 and n_sup * _NSUP == n_ch_w
    assert rows_w % 8 == 0 and nz * _ZR == rows_w
    mesh = plsc.VectorSubcoreMesh(core_axis_name="c", subcore_axis_name="s")

    scratch = [
        pltpu.VMEM((_NSUP, _CH), jnp.int32),
        pltpu.VMEM((_NSUP, _CH), jnp.int32),
        pltpu.VMEM((_NSUP, _CH), jnp.int32),
        pltpu.VMEM((_CH, dh), _F32),
        pltpu.VMEM((_ZR, dh), _F32),
        pltpu.VMEM_SHARED((n_pad, dh), _F32),
        pltpu.SemaphoreType.DMA,
    ]

    @functools.partial(pl.kernel,
                       out_type=jax.ShapeDtypeStruct((nc, n_pad, dh), _F32),
                       mesh=mesh, scratch_types=scratch,
                       compiler_params=pltpu.CompilerParams(
                           use_tc_tiling_on_sc=False))
    def k(g_hbm, src_hbm, dst_hbm, out_hbm, sraw, sadj, dstv, rowb, zb, acc,
          sem):
        c = lax.axis_index("c")
        s = lax.axis_index("s")

        def _zb(i, carry):
            for t in range(dh // 16):
                zb[i, pl.ds(t * 16, 16)] = jnp.zeros((16,), _F32)
            return carry
        lax.fori_loop(0, _ZR, _zb, 0)

        r0 = s * rows_w

        def _za(i, carry):
            pltpu.sync_copy(zb, acc.at[pl.ds(r0 + i * _ZR, _ZR)])
            return carry
        lax.fori_loop(0, nz, _za, 0)
        plsc.subcore_barrier()

        base_ch = s * n_ch_w if column_split else (s * nc + c) * n_ch_w

        def _sup(k0, carry):
            row = base_ch + k0 * _NSUP
            pltpu.sync_copy(src_hbm.at[pl.ds(row, _NSUP)], sraw)
            pltpu.sync_copy(dst_hbm.at[pl.ds(row, _NSUP)], dstv)
            if column_split:
                def _adj(i, cc):
                    for t in range(_CH // 16):
                        v = sraw[i, pl.ds(t * 16, 16)]
                        sadj[i, pl.ds(t * 16, 16)] = v * 2 + c
                    return cc
                lax.fori_loop(0, _NSUP, _adj, 0)
            idxs = sadj if column_split else sraw

            def _inner(j, cc):
                pltpu.async_copy(g_hbm.at[idxs.at[j]], rowb, sem).wait()
                pltpu.sync_copy(rowb, acc.at[dstv.at[j]], add=True)
                return cc
            lax.fori_loop(0, _NSUP, _inner, 0)
            return carry
        lax.fori_loop(0, n_sup, _sup, 0)
        plsc.subcore_barrier()

        def _wb(i, carry):
            pltpu.sync_copy(acc.at[pl.ds(r0 + i * _ZR, _ZR)],
                            out_hbm.at[c].at[pl.ds(r0 + i * _ZR, _ZR)])
            return carry
        lax.fori_loop(0, nz, _wb, 0)

    return k(gflat, src2, dst2)


def _ceil_to(v, m):
    return ((v + m - 1) // m) * m


def kernel(x, edge_index, params):
    n = x.shape[0]
    e = edge_index.shape[1]
    src = edge_index[0]
    dst = edge_index[1]

    prm = dict(params)
    for k in list(prm):
        if k.endswith("_b"):
            prm[k] = prm[k].reshape(1, -1)

    ep = _ceil_to(e, 32 * _CH * 8)
    n_pad = _ceil_to(n + 1, 16 * _ZR)
    p = ep - e
    pad_ar = jnp.arange(p, dtype=jnp.int32)
    srcp = jnp.concatenate([src, pad_ar % n]).reshape(ep // _CH, _CH)
    dstp = jnp.concatenate([dst, n + pad_ar % (n_pad - n)]).reshape(
        ep // _CH, _CH)

    p1 = _pass_a(x, prm, n)
    agg1 = _sc_segsum(x.reshape(2 * n, 32), srcp, dstp, n_pad, 32, True)
    p2, u = _pass_bc(p1, agg1, prm, n, 2)
    agg2 = _sc_segsum(u.reshape(2 * n, 16), srcp, dstp, n_pad, 16, True)
    p3, v = _pass_bc(p2, agg2, prm, n, 3)
    agg3 = _sc_segsum(v, srcp, dstp, n_pad, 16, False)
    return _pass_d(p3, agg3, prm, n)


# speedup vs baseline: 12.4530x; 1.8856x over previous
import functools

import jax
import jax.numpy as jnp
from jax import lax
from jax.experimental import pallas as pl
from jax.experimental.pallas import tpu as pltpu
from jax.experimental.pallas import tpu_sc as plsc

_F32 = jnp.float32
_BLK = 2000


def _full_spec(shape):
    nd = len(shape)
    return pl.BlockSpec(shape, lambda i, _nd=nd: (0,) * _nd)


def _row_spec(f, blk=_BLK):
    return pl.BlockSpec((blk, f), lambda i: (i, 0))


def _agg_spec(f, blk=_BLK):
    return pl.BlockSpec((2, blk, f), lambda i: (0, i, 0))


def _mm(v, w_ref, b_ref):
    return jnp.dot(v, w_ref[...], preferred_element_type=_F32) + b_ref[...]


def _pa_body(x, w11, b11, w12, b12, w13, b13, p):
    xb = x[...]
    a = jnp.maximum(_mm(xb, w11, b11), 0.0)
    m = _mm(xb, w12, b12) * _mm(xb, w13, b13)
    p[...] = a + jnp.maximum(m, 0.0)


def _pbc_body(p_in, agg, wc, bc, wg1, bg1, wg2, bg2, w1, b1, w2, b2, w3, b3,
              p_out, u_out):
    a = agg[...]
    aggf = jnp.concatenate([a[0], a[1]], axis=-1)
    h = p_in[...] + jnp.maximum(_mm(aggf, wc, bc), 0.0)
    u = jax.nn.sigmoid(_mm(h, wg1, bg1)) * _mm(h, wg2, bg2)
    u_out[...] = u
    p_out[...] = (jnp.maximum(_mm(u, w1, b1), 0.0)
                  + jnp.maximum(_mm(u, w2, b2) * _mm(u, w3, b3), 0.0))


def _pd_body(p_in, agg, wc, bc, wf, bf, out):
    a = agg[...]
    aggf = a[0] + a[1]
    h = p_in[...] + jnp.maximum(_mm(aggf, wc, bc), 0.0)
    out[...] = _mm(h, wf, bf)


def _pass_a(x, prm, n):
    names = ["fc11_W", "fc11_b", "fc12_W", "fc12_b", "fc13_W", "fc13_b"]
    args = [prm[k] for k in names]
    return pl.pallas_call(
        _pa_body,
        grid=(n // _BLK,),
        in_specs=[_row_spec(64)] + [_full_spec(a.shape) for a in args],
        out_specs=_row_spec(64),
        out_shape=jax.ShapeDtypeStruct((n, 64), _F32),
    )(x, *args)


def _pass_bc(p_in, agg, prm, n, stage):
    if stage == 2:
        fi, fo = 64, 32
        names = ["conv11_W", "conv11_b", "gate1_fc1_W", "gate1_fc1_b",
                 "gate1_fc2_W", "gate1_fc2_b", "fc21_W", "fc21_b",
                 "fc22_W", "fc22_b", "fc23_W", "fc23_b"]
    else:
        fi, fo = 32, 16
        names = ["conv21_W", "conv21_b", "gate2_fc1_W", "gate2_fc1_b",
                 "gate2_fc2_W", "gate2_fc2_b", "fc31_W", "fc31_b",
                 "fc32_W", "fc32_b", "fc33_W", "fc33_b"]
    args = [prm[k] for k in names]
    return pl.pallas_call(
        _pbc_body,
        grid=(n // _BLK,),
        in_specs=([_row_spec(fi), _agg_spec(fi // 2)]
                  + [_full_spec(a.shape) for a in args]),
        out_specs=[_row_spec(fo), _row_spec(fo)],
        out_shape=[jax.ShapeDtypeStruct((n, fo), _F32)] * 2,
    )(p_in, agg, *args)


def _pass_d(p_in, agg, prm, n):
    names = ["conv31_W", "conv31_b", "fc2_W", "fc2_b"]
    args = [prm[k] for k in names]
    return pl.pallas_call(
        _pd_body,
        grid=(n // _BLK,),
        in_specs=([_row_spec(16), _agg_spec(16)]
                  + [_full_spec(a.shape) for a in args]),
        out_specs=_row_spec(16),
        out_shape=jax.ShapeDtypeStruct((n, 16), _F32),
    )(p_in, agg, *args)


_CH = 128
_NSUP = 16
_ZR = 112


def _sc_segsum(gflat, src2, dst2, n_pad, dh, column_split):
    e_chunks, ch = src2.shape
    assert ch == _CH
    nc, ns = 2, 16
    n_ch_w = e_chunks // ns if column_split else e_chunks // (ns * nc)
    nsup = _NSUP if n_ch_w % _NSUP == 0 else 8
    n_sup = n_ch_w // nsup
    nbuf = min(nsup, 4 if dh > 16 else 8)
    rows_w = n_pad // ns
    nz = rows_w // _ZR
    assert n_ch_w % 8 == 0 and n_sup * nsup == n_ch_w
    assert rows_w % 8 == 0 and nz * _ZR == rows_w
    mesh = plsc.VectorSubcoreMesh(core_axis_name="c", subcore_axis_name="s")

    scratch = [
        pltpu.VMEM((nsup, _CH), jnp.int32),
        pltpu.VMEM((nsup, _CH), jnp.int32),
        pltpu.VMEM((nsup, _CH), jnp.int32),
        pltpu.VMEM((nbuf, _CH, dh), _F32),
        pltpu.VMEM((_ZR, dh), _F32),
        pltpu.VMEM_SHARED((n_pad, dh), _F32),
        [pltpu.SemaphoreType.DMA] * nbuf,
        [pltpu.SemaphoreType.DMA] * nbuf,
    ]

    @functools.partial(pl.kernel,
                       out_type=jax.ShapeDtypeStruct((nc, n_pad, dh), _F32),
                       mesh=mesh, scratch_types=scratch,
                       compiler_params=pltpu.CompilerParams(
                           use_tc_tiling_on_sc=False))
    def k(g_hbm, src_hbm, dst_hbm, out_hbm, sraw, sadj, dstv, rowb, zb, acc,
          gsems, ssems):
        c = lax.axis_index("c")
        s = lax.axis_index("s")

        def _zb(i, carry):
            for t in range(dh // 16):
                zb[i, pl.ds(t * 16, 16)] = jnp.zeros((16,), _F32)
            return carry
        lax.fori_loop(0, _ZR, _zb, 0)

        r0 = s * rows_w

        def _za(i, carry):
            pltpu.sync_copy(zb, acc.at[pl.ds(r0 + i * _ZR, _ZR)])
            return carry
        lax.fori_loop(0, nz, _za, 0)
        plsc.subcore_barrier()

        base_ch = s * n_ch_w if column_split else (s * nc + c) * n_ch_w

        def _sup(k0, carry):
            row = base_ch + k0 * nsup
            pltpu.sync_copy(src_hbm.at[pl.ds(row, nsup)], sraw)
            pltpu.sync_copy(dst_hbm.at[pl.ds(row, nsup)], dstv)
            if column_split:
                def _adj(i, cc):
                    for t in range(_CH // 16):
                        v = sraw[i, pl.ds(t * 16, 16)]
                        sadj[i, pl.ds(t * 16, 16)] = v * 2 + c
                    return cc
                lax.fori_loop(0, nsup, _adj, 0)
            idxs = sadj if column_split else sraw

            gds = {b: pltpu.async_copy(g_hbm.at[idxs.at[b]], rowb.at[b],
                                       gsems[b]) for b in range(nbuf)}
            tail = []
            for b in range(nsup):
                bb = b % nbuf
                gds[b].wait()
                sd = pltpu.async_copy(rowb.at[bb], acc.at[dstv.at[b]],
                                      ssems[bb], add=True)
                nb = b + nbuf
                if nb < nsup:
                    sd.wait()
                    gds[nb] = pltpu.async_copy(g_hbm.at[idxs.at[nb]],
                                               rowb.at[bb], gsems[bb])
                else:
                    tail.append(sd)
            for sd in tail:
                sd.wait()
            return carry
        lax.fori_loop(0, n_sup, _sup, 0)
        plsc.subcore_barrier()

        def _wb(i, carry):
            pltpu.sync_copy(acc.at[pl.ds(r0 + i * _ZR, _ZR)],
                            out_hbm.at[c].at[pl.ds(r0 + i * _ZR, _ZR)])
            return carry
        lax.fori_loop(0, nz, _wb, 0)

    return k(gflat, src2, dst2)


def _ceil_to(v, m):
    return ((v + m - 1) // m) * m


def kernel(x, edge_index, params):
    n = x.shape[0]
    e = edge_index.shape[1]
    src = edge_index[0]
    dst = edge_index[1]

    prm = dict(params)
    for k in list(prm):
        if k.endswith("_b"):
            prm[k] = prm[k].reshape(1, -1)

    ep = _ceil_to(e, 32 * _CH * 8)
    n_pad = _ceil_to(n + 1, 16 * _ZR)
    p = ep - e
    pad_ar = jnp.arange(p, dtype=jnp.int32)
    srcp = jnp.concatenate([src, pad_ar % n]).reshape(ep // _CH, _CH)
    dstp = jnp.concatenate([dst, n + pad_ar % (n_pad - n)]).reshape(
        ep // _CH, _CH)

    p1 = _pass_a(x, prm, n)
    agg1 = _sc_segsum(x.reshape(2 * n, 32), srcp, dstp, n_pad, 32, True)
    p2, u = _pass_bc(p1, agg1, prm, n, 2)
    agg2 = _sc_segsum(u.reshape(2 * n, 16), srcp, dstp, n_pad, 16, True)
    p3, v = _pass_bc(p2, agg2, prm, n, 3)
    agg3 = _sc_segsum(v, srcp, dstp, n_pad, 16, False)
    return _pass_d(p3, agg3, prm, n)


# speedup vs baseline: 12.4683x; 1.0012x over previous
import functools

import jax
import jax.numpy as jnp
from jax import lax
from jax.experimental import pallas as pl
from jax.experimental.pallas import tpu as pltpu
from jax.experimental.pallas import tpu_sc as plsc

_F32 = jnp.float32
_BLK = 2000


def _full_spec(shape):
    nd = len(shape)
    return pl.BlockSpec(shape, lambda i, _nd=nd: (0,) * _nd)


def _row_spec(f, blk=_BLK):
    return pl.BlockSpec((blk, f), lambda i: (i, 0))


def _agg_spec(f, blk=_BLK):
    return pl.BlockSpec((2, blk, f), lambda i: (0, i, 0))


def _mm(v, w_ref, b_ref):
    return jnp.dot(v, w_ref[...], preferred_element_type=_F32) + b_ref[...]


def _pa_body(x, w11, b11, w12, b12, w13, b13, p):
    xb = x[...]
    a = jnp.maximum(_mm(xb, w11, b11), 0.0)
    m = _mm(xb, w12, b12) * _mm(xb, w13, b13)
    p[...] = a + jnp.maximum(m, 0.0)


def _pbc_body(p_in, agg, wc, bc, wg1, bg1, wg2, bg2, w1, b1, w2, b2, w3, b3,
              p_out, u_out):
    a = agg[...]
    aggf = jnp.concatenate([a[0], a[1]], axis=-1)
    h = p_in[...] + jnp.maximum(_mm(aggf, wc, bc), 0.0)
    u = jax.nn.sigmoid(_mm(h, wg1, bg1)) * _mm(h, wg2, bg2)
    u_out[...] = u
    p_out[...] = (jnp.maximum(_mm(u, w1, b1), 0.0)
                  + jnp.maximum(_mm(u, w2, b2) * _mm(u, w3, b3), 0.0))


def _pd_body(p_in, agg, wc, bc, wf, bf, out):
    a = agg[...]
    aggf = a[0] + a[1]
    h = p_in[...] + jnp.maximum(_mm(aggf, wc, bc), 0.0)
    out[...] = _mm(h, wf, bf)


def _pass_a(x, prm, n):
    names = ["fc11_W", "fc11_b", "fc12_W", "fc12_b", "fc13_W", "fc13_b"]
    args = [prm[k] for k in names]
    return pl.pallas_call(
        _pa_body,
        grid=(n // _BLK,),
        in_specs=[_row_spec(64)] + [_full_spec(a.shape) for a in args],
        out_specs=_row_spec(64),
        out_shape=jax.ShapeDtypeStruct((n, 64), _F32),
    )(x, *args)


def _pass_bc(p_in, agg, prm, n, stage):
    if stage == 2:
        fi, fo = 64, 32
        names = ["conv11_W", "conv11_b", "gate1_fc1_W", "gate1_fc1_b",
                 "gate1_fc2_W", "gate1_fc2_b", "fc21_W", "fc21_b",
                 "fc22_W", "fc22_b", "fc23_W", "fc23_b"]
    else:
        fi, fo = 32, 16
        names = ["conv21_W", "conv21_b", "gate2_fc1_W", "gate2_fc1_b",
                 "gate2_fc2_W", "gate2_fc2_b", "fc31_W", "fc31_b",
                 "fc32_W", "fc32_b", "fc33_W", "fc33_b"]
    args = [prm[k] for k in names]
    return pl.pallas_call(
        _pbc_body,
        grid=(n // _BLK,),
        in_specs=([_row_spec(fi), _agg_spec(fi // 2)]
                  + [_full_spec(a.shape) for a in args]),
        out_specs=[_row_spec(fo), _row_spec(fo)],
        out_shape=[jax.ShapeDtypeStruct((n, fo), _F32)] * 2,
    )(p_in, agg, *args)


def _pass_d(p_in, agg, prm, n):
    names = ["conv31_W", "conv31_b", "fc2_W", "fc2_b"]
    args = [prm[k] for k in names]
    return pl.pallas_call(
        _pd_body,
        grid=(n // _BLK,),
        in_specs=([_row_spec(16), _agg_spec(16)]
                  + [_full_spec(a.shape) for a in args]),
        out_specs=_row_spec(16),
        out_shape=jax.ShapeDtypeStruct((n, 16), _F32),
    )(p_in, agg, *args)


_CH = 128
_NSUP = 16
_ZR = 112


def _sc_segsum(gflat, src2, dst2, n_pad, dh, column_split):
    e_chunks, ch = src2.shape
    assert ch == _CH
    nc, ns = 2, 16
    n_ch_w = e_chunks // ns if column_split else e_chunks // (ns * nc)
    nsup = _NSUP if n_ch_w % _NSUP == 0 else 8
    n_sup = n_ch_w // nsup
    nbuf = min(nsup, 4 if dh > 16 else 8)
    rows_w = n_pad // ns
    nz = rows_w // _ZR
    assert n_ch_w % 8 == 0 and n_sup * nsup == n_ch_w
    assert rows_w % 8 == 0 and nz * _ZR == rows_w
    mesh = plsc.VectorSubcoreMesh(core_axis_name="c", subcore_axis_name="s")

    scratch = [
        pltpu.VMEM((nsup, _CH), jnp.int32),
        pltpu.VMEM((nsup, _CH), jnp.int32),
        pltpu.VMEM((nsup, _CH), jnp.int32),
        pltpu.VMEM((nbuf, _CH, dh), _F32),
        pltpu.VMEM((_ZR, dh), _F32),
        pltpu.VMEM_SHARED((n_pad, dh), _F32),
        [pltpu.SemaphoreType.DMA] * nbuf,
        [pltpu.SemaphoreType.DMA] * nbuf,
    ]

    @functools.partial(pl.kernel,
                       out_type=jax.ShapeDtypeStruct((nc, n_pad, dh), _F32),
                       mesh=mesh, scratch_types=scratch,
                       compiler_params=pltpu.CompilerParams(
                           use_tc_tiling_on_sc=False))
    def k(g_hbm, src_hbm, dst_hbm, out_hbm, sraw, sadj, dstv, rowb, zb, acc,
          gsems, ssems):
        c = lax.axis_index("c")
        s = lax.axis_index("s")

        def _zb(i, carry):
            for t in range(dh // 16):
                zb[i, pl.ds(t * 16, 16)] = jnp.zeros((16,), _F32)
            return carry
        lax.fori_loop(0, _ZR, _zb, 0)

        r0 = s * rows_w

        def _za(i, carry):
            pltpu.sync_copy(zb, acc.at[pl.ds(r0 + i * _ZR, _ZR)])
            return carry
        lax.fori_loop(0, nz, _za, 0)
        plsc.subcore_barrier()

        base_ch = s * n_ch_w if column_split else (s * nc + c) * n_ch_w

        def _sup(k0, carry):
            row = base_ch + k0 * nsup
            pltpu.sync_copy(src_hbm.at[pl.ds(row, nsup)], sraw)
            pltpu.sync_copy(dst_hbm.at[pl.ds(row, nsup)], dstv)
            if column_split:
                def _adj(i, cc):
                    for t in range(_CH // 16):
                        v = sraw[i, pl.ds(t * 16, 16)]
                        sadj[i, pl.ds(t * 16, 16)] = v * 2 + c
                    return cc
                lax.fori_loop(0, nsup, _adj, 0)
            idxs = sadj if column_split else sraw

            gds = {b: pltpu.async_copy(g_hbm.at[idxs.at[b]], rowb.at[b],
                                       gsems[b]) for b in range(nbuf)}
            sds = {}
            for b in range(nsup):
                pb = b - 1
                if pb >= 0 and pb + nbuf < nsup:
                    sds[pb].wait()
                    pbb = pb % nbuf
                    gds[pb + nbuf] = pltpu.async_copy(
                        g_hbm.at[idxs.at[pb + nbuf]], rowb.at[pbb],
                        gsems[pbb])
                gds[b].wait()
                sds[b] = pltpu.async_copy(rowb.at[b % nbuf],
                                          acc.at[dstv.at[b]],
                                          ssems[b % nbuf], add=True)
            for b in range(max(0, nsup - nbuf), nsup):
                sds[b].wait()
            return carry
        lax.fori_loop(0, n_sup, _sup, 0)
        plsc.subcore_barrier()

        def _wb(i, carry):
            pltpu.sync_copy(acc.at[pl.ds(r0 + i * _ZR, _ZR)],
                            out_hbm.at[c].at[pl.ds(r0 + i * _ZR, _ZR)])
            return carry
        lax.fori_loop(0, nz, _wb, 0)

    return k(gflat, src2, dst2)


def _ceil_to(v, m):
    return ((v + m - 1) // m) * m


def kernel(x, edge_index, params):
    n = x.shape[0]
    e = edge_index.shape[1]
    src = edge_index[0]
    dst = edge_index[1]

    prm = dict(params)
    for k in list(prm):
        if k.endswith("_b"):
            prm[k] = prm[k].reshape(1, -1)

    ep = _ceil_to(e, 32 * _CH * 8)
    n_pad = _ceil_to(n + 1, 16 * _ZR)
    p = ep - e
    pad_ar = jnp.arange(p, dtype=jnp.int32)
    srcp = jnp.concatenate([src, pad_ar % n]).reshape(ep // _CH, _CH)
    dstp = jnp.concatenate([dst, n + pad_ar % (n_pad - n)]).reshape(
        ep // _CH, _CH)

    p1 = _pass_a(x, prm, n)
    agg1 = _sc_segsum(x.reshape(2 * n, 32), srcp, dstp, n_pad, 32, True)
    p2, u = _pass_bc(p1, agg1, prm, n, 2)
    agg2 = _sc_segsum(u.reshape(2 * n, 16), srcp, dstp, n_pad, 16, True)
    p3, v = _pass_bc(p2, agg2, prm, n, 3)
    agg3 = _sc_segsum(v, srcp, dstp, n_pad, 16, False)
    return _pass_d(p3, agg3, prm, n)


# speedup vs baseline: 12.5574x; 1.0071x over previous
import functools

import jax
import jax.numpy as jnp
from jax import lax
from jax.experimental import pallas as pl
from jax.experimental.pallas import tpu as pltpu
from jax.experimental.pallas import tpu_sc as plsc

_F32 = jnp.float32
_BLK = 2000


def _full_spec(shape):
    nd = len(shape)
    return pl.BlockSpec(shape, lambda i, _nd=nd: (0,) * _nd)


def _row_spec(f, blk=_BLK):
    return pl.BlockSpec((blk, f), lambda i: (i, 0))


def _agg_spec(f, blk=_BLK):
    return pl.BlockSpec((2, blk, f), lambda i: (0, i, 0))


def _mm(v, w_ref, b_ref):
    return jnp.dot(v, w_ref[...], preferred_element_type=_F32) + b_ref[...]


def _pa_body(x, w11, b11, w12, b12, w13, b13, p):
    xb = x[...]
    a = jnp.maximum(_mm(xb, w11, b11), 0.0)
    m = _mm(xb, w12, b12) * _mm(xb, w13, b13)
    p[...] = a + jnp.maximum(m, 0.0)


def _pbc_body(p_in, agg, wc, bc, wg1, bg1, wg2, bg2, w1, b1, w2, b2, w3, b3,
              m_out):
    fi = p_in.shape[1]
    h = p_in[...] + jnp.maximum(_mm(agg[...], wc, bc), 0.0)
    u = jax.nn.sigmoid(_mm(h, wg1, bg1)) * _mm(h, wg2, bg2)
    p = (jnp.maximum(_mm(u, w1, b1), 0.0)
         + jnp.maximum(_mm(u, w2, b2) * _mm(u, w3, b3), 0.0))
    m_out[...] = jnp.concatenate([p, u], axis=-1)


def _pbc_merged_body(m_in, agg, wc, bc, wg1, bg1, wg2, bg2, w1, b1, w2, b2,
                     w3, b3, m_out):
    m = m_in[...]
    fi = m.shape[1] // 2
    h = m[:, :fi] + jnp.maximum(_mm(agg[...], wc, bc), 0.0)
    u = jax.nn.sigmoid(_mm(h, wg1, bg1)) * _mm(h, wg2, bg2)
    p = (jnp.maximum(_mm(u, w1, b1), 0.0)
         + jnp.maximum(_mm(u, w2, b2) * _mm(u, w3, b3), 0.0))
    m_out[...] = jnp.concatenate([p, u], axis=-1)


def _pd_body(m_in, agg, wc, bc, wf, bf, out):
    m = m_in[...]
    fo = m.shape[1] // 2
    a = agg[...]
    aggf = a[:, :fo] + a[:, fo:]
    h = m[:, :fo] + jnp.maximum(_mm(aggf, wc, bc), 0.0)
    out[...] = _mm(h, wf, bf)


def _pass_a(x, prm, n):
    names = ["fc11_W", "fc11_b", "fc12_W", "fc12_b", "fc13_W", "fc13_b"]
    args = [prm[k] for k in names]
    return pl.pallas_call(
        _pa_body,
        grid=(n // _BLK,),
        in_specs=[_row_spec(64)] + [_full_spec(a.shape) for a in args],
        out_specs=_row_spec(64),
        out_shape=jax.ShapeDtypeStruct((n, 64), _F32),
    )(x, *args)


def _pass_bc(p_in, agg, prm, n, stage):
    if stage == 2:
        fi, fo = 64, 32
        names = ["conv11_W", "conv11_b", "gate1_fc1_W", "gate1_fc1_b",
                 "gate1_fc2_W", "gate1_fc2_b", "fc21_W", "fc21_b",
                 "fc22_W", "fc22_b", "fc23_W", "fc23_b"]
    else:
        fi, fo = 32, 16
        names = ["conv21_W", "conv21_b", "gate2_fc1_W", "gate2_fc1_b",
                 "gate2_fc2_W", "gate2_fc2_b", "fc31_W", "fc31_b",
                 "fc32_W", "fc32_b", "fc33_W", "fc33_b"]
    args = [prm[k] for k in names]
    body = _pbc_body if p_in.shape[1] == fi else _pbc_merged_body
    return pl.pallas_call(
        body,
        grid=(n // _BLK,),
        in_specs=([_row_spec(p_in.shape[1]), _row_spec(fi)]
                  + [_full_spec(a.shape) for a in args]),
        out_specs=_row_spec(2 * fo),
        out_shape=jax.ShapeDtypeStruct((n, 2 * fo), _F32),
    )(p_in, agg, *args)


def _pass_d(p_in, agg, prm, n):
    names = ["conv31_W", "conv31_b", "fc2_W", "fc2_b"]
    args = [prm[k] for k in names]
    return pl.pallas_call(
        _pd_body,
        grid=(n // _BLK,),
        in_specs=([_row_spec(p_in.shape[1]), _row_spec(32)]
                  + [_full_spec(a.shape) for a in args]),
        out_specs=_row_spec(16),
        out_shape=jax.ShapeDtypeStruct((n, 16), _F32),
    )(p_in, agg, *args)


_CH = 128
_NSUP = 16
_ZR = 112


def _sc_segsum(gflat, src2, dst2, n_pad, dh, column_split):
    e_chunks, ch = src2.shape
    assert ch == _CH
    nc, ns = 2, 16
    n_ch_w = e_chunks // ns if column_split else e_chunks // (ns * nc)
    nsup = _NSUP if n_ch_w % _NSUP == 0 else 8
    n_sup = n_ch_w // nsup
    nbuf = min(nsup, 4 if dh > 16 else 8)
    rows_w = n_pad // ns
    nz = rows_w // _ZR
    assert n_ch_w % 8 == 0 and n_sup * nsup == n_ch_w
    assert rows_w % 8 == 0 and nz * _ZR == rows_w
    mesh = plsc.VectorSubcoreMesh(core_axis_name="c", subcore_axis_name="s")

    scratch = [
        pltpu.VMEM((nsup, _CH), jnp.int32),
        pltpu.VMEM((nsup, _CH), jnp.int32),
        pltpu.VMEM((nsup, _CH), jnp.int32),
        pltpu.VMEM((nbuf, _CH, dh), _F32),
        pltpu.VMEM((_ZR, dh), _F32),
        pltpu.VMEM_SHARED((n_pad, dh), _F32),
        [pltpu.SemaphoreType.DMA] * nbuf,
        [pltpu.SemaphoreType.DMA] * nbuf,
    ]

    @functools.partial(pl.kernel,
                       out_type=jax.ShapeDtypeStruct((n_pad, nc * dh), _F32),
                       mesh=mesh, scratch_types=scratch,
                       compiler_params=pltpu.CompilerParams(
                           use_tc_tiling_on_sc=False))
    def k(g_hbm, src_hbm, dst_hbm, out_hbm, sraw, sadj, dstv, rowb, zb, acc,
          gsems, ssems):
        c = lax.axis_index("c")
        s = lax.axis_index("s")

        def _zb(i, carry):
            for t in range(dh // 16):
                zb[i, pl.ds(t * 16, 16)] = jnp.zeros((16,), _F32)
            return carry
        lax.fori_loop(0, _ZR, _zb, 0)

        r0 = s * rows_w

        def _za(i, carry):
            pltpu.sync_copy(zb, acc.at[pl.ds(r0 + i * _ZR, _ZR)])
            return carry
        lax.fori_loop(0, nz, _za, 0)
        plsc.subcore_barrier()

        base_ch = s * n_ch_w if column_split else (s * nc + c) * n_ch_w

        def _sup(k0, carry):
            row = base_ch + k0 * nsup
            pltpu.sync_copy(src_hbm.at[pl.ds(row, nsup)], sraw)
            pltpu.sync_copy(dst_hbm.at[pl.ds(row, nsup)], dstv)
            if column_split:
                def _adj(i, cc):
                    for t in range(_CH // 16):
                        v = sraw[i, pl.ds(t * 16, 16)]
                        sadj[i, pl.ds(t * 16, 16)] = v * 2 + c
                    return cc
                lax.fori_loop(0, nsup, _adj, 0)
            idxs = sadj if column_split else sraw

            gds = {b: pltpu.async_copy(g_hbm.at[idxs.at[b]], rowb.at[b],
                                       gsems[b]) for b in range(nbuf)}
            sds = {}
            for b in range(nsup):
                pb = b - 1
                if pb >= 0 and pb + nbuf < nsup:
                    sds[pb].wait()
                    pbb = pb % nbuf
                    gds[pb + nbuf] = pltpu.async_copy(
                        g_hbm.at[idxs.at[pb + nbuf]], rowb.at[pbb],
                        gsems[pbb])
                gds[b].wait()
                sds[b] = pltpu.async_copy(rowb.at[b % nbuf],
                                          acc.at[dstv.at[b]],
                                          ssems[b % nbuf], add=True)
            for b in range(max(0, nsup - nbuf), nsup):
                sds[b].wait()
            return carry
        lax.fori_loop(0, n_sup, _sup, 0)
        plsc.subcore_barrier()

        def _wb(i, carry):
            pltpu.sync_copy(acc.at[pl.ds(r0 + i * _ZR, _ZR)],
                            out_hbm.at[pl.ds(r0 + i * _ZR, _ZR),
                                       pl.ds(c * dh, dh)])
            return carry
        lax.fori_loop(0, nz, _wb, 0)

    return k(gflat, src2, dst2)


def _ceil_to(v, m):
    return ((v + m - 1) // m) * m


def kernel(x, edge_index, params):
    n = x.shape[0]
    e = edge_index.shape[1]
    src = edge_index[0]
    dst = edge_index[1]

    prm = dict(params)
    for k in list(prm):
        if k.endswith("_b"):
            prm[k] = prm[k].reshape(1, -1)

    ep = _ceil_to(e, 32 * _CH * 8)
    n_pad = _ceil_to(n + 1, 16 * _ZR)
    p = ep - e
    pad_ar = jnp.arange(p, dtype=jnp.int32)
    srcp = jnp.concatenate([src, pad_ar % n]).reshape(ep // _CH, _CH)
    dstp = jnp.concatenate([dst, n + pad_ar % (n_pad - n)]).reshape(
        ep // _CH, _CH)

    p1 = _pass_a(x, prm, n)
    agg1 = _sc_segsum(x.reshape(2 * n, 32), srcp, dstp, n_pad, 32, True)
    m2 = _pass_bc(p1, agg1, prm, n, 2)
    u = m2[:, 32:]
    agg2 = _sc_segsum(u.reshape(2 * n, 16), srcp, dstp, n_pad, 16, True)
    m3 = _pass_bc(m2, agg2, prm, n, 3)
    v = m3[:, 16:]
    agg3 = _sc_segsum(v, srcp, dstp, n_pad, 16, False)
    return _pass_d(m3, agg3, prm, n)


# speedup vs baseline: 13.1270x; 1.0454x over previous
import functools

import jax
import jax.numpy as jnp
from jax import lax
from jax.experimental import pallas as pl
from jax.experimental.pallas import tpu as pltpu
from jax.experimental.pallas import tpu_sc as plsc

_F32 = jnp.float32
_BLK = 2000


def _full_spec(shape):
    nd = len(shape)
    return pl.BlockSpec(shape, lambda i, _nd=nd: (0,) * _nd)


def _row_spec(f, blk=_BLK):
    return pl.BlockSpec((blk, f), lambda i: (i, 0))


def _agg_spec(f, blk=_BLK):
    return pl.BlockSpec((2, blk, f), lambda i: (0, i, 0))


def _mm(v, w_ref, b_ref):
    return jnp.dot(v, w_ref[...], preferred_element_type=_F32) + b_ref[...]


def _pa_body(x, w11, b11, w12, b12, w13, b13, p):
    xb = x[...]
    a = jnp.maximum(_mm(xb, w11, b11), 0.0)
    m = _mm(xb, w12, b12) * _mm(xb, w13, b13)
    p[...] = a + jnp.maximum(m, 0.0)


def _pbc_body(p_in, agg, wc, bc, wg1, bg1, wg2, bg2, w1, b1, w2, b2, w3, b3,
              m_out):
    fi = p_in.shape[1]
    h = p_in[...] + jnp.maximum(_mm(agg[...], wc, bc), 0.0)
    u = jax.nn.sigmoid(_mm(h, wg1, bg1)) * _mm(h, wg2, bg2)
    p = (jnp.maximum(_mm(u, w1, b1), 0.0)
         + jnp.maximum(_mm(u, w2, b2) * _mm(u, w3, b3), 0.0))
    m_out[...] = jnp.concatenate([p, u], axis=-1)


def _pbc_merged_body(m_in, agg, wc, bc, wg1, bg1, wg2, bg2, w1, b1, w2, b2,
                     w3, b3, m_out):
    m = m_in[...]
    fi = m.shape[1] // 2
    h = m[:, :fi] + jnp.maximum(_mm(agg[...], wc, bc), 0.0)
    u = jax.nn.sigmoid(_mm(h, wg1, bg1)) * _mm(h, wg2, bg2)
    p = (jnp.maximum(_mm(u, w1, b1), 0.0)
         + jnp.maximum(_mm(u, w2, b2) * _mm(u, w3, b3), 0.0))
    m_out[...] = jnp.concatenate([p, u], axis=-1)


def _pd_body(m_in, agg, wc, bc, wf, bf, out):
    m = m_in[...]
    fo = m.shape[1] // 2
    a = agg[...]
    aggf = a[:, :fo] + a[:, fo:]
    h = m[:, :fo] + jnp.maximum(_mm(aggf, wc, bc), 0.0)
    out[...] = _mm(h, wf, bf)


def _pass_a(x, prm, n):
    names = ["fc11_W", "fc11_b", "fc12_W", "fc12_b", "fc13_W", "fc13_b"]
    args = [prm[k] for k in names]
    return pl.pallas_call(
        _pa_body,
        grid=(n // _BLK,),
        in_specs=[_row_spec(64)] + [_full_spec(a.shape) for a in args],
        out_specs=_row_spec(64),
        out_shape=jax.ShapeDtypeStruct((n, 64), _F32),
    )(x, *args)


def _pass_bc(p_in, agg, prm, n, stage):
    if stage == 2:
        fi, fo = 64, 32
        names = ["conv11_W", "conv11_b", "gate1_fc1_W", "gate1_fc1_b",
                 "gate1_fc2_W", "gate1_fc2_b", "fc21_W", "fc21_b",
                 "fc22_W", "fc22_b", "fc23_W", "fc23_b"]
    else:
        fi, fo = 32, 16
        names = ["conv21_W", "conv21_b", "gate2_fc1_W", "gate2_fc1_b",
                 "gate2_fc2_W", "gate2_fc2_b", "fc31_W", "fc31_b",
                 "fc32_W", "fc32_b", "fc33_W", "fc33_b"]
    args = [prm[k] for k in names]
    body = _pbc_body if p_in.shape[1] == fi else _pbc_merged_body
    return pl.pallas_call(
        body,
        grid=(n // _BLK,),
        in_specs=([_row_spec(p_in.shape[1]), _row_spec(fi)]
                  + [_full_spec(a.shape) for a in args]),
        out_specs=_row_spec(2 * fo),
        out_shape=jax.ShapeDtypeStruct((n, 2 * fo), _F32),
    )(p_in, agg, *args)


def _pass_d(p_in, agg, prm, n):
    names = ["conv31_W", "conv31_b", "fc2_W", "fc2_b"]
    args = [prm[k] for k in names]
    return pl.pallas_call(
        _pd_body,
        grid=(n // _BLK,),
        in_specs=([_row_spec(p_in.shape[1]), _row_spec(32)]
                  + [_full_spec(a.shape) for a in args]),
        out_specs=_row_spec(16),
        out_shape=jax.ShapeDtypeStruct((n, 16), _F32),
    )(p_in, agg, *args)


_CH = 128
_NSUP = 16
_ZR = 112


def _sc_segsum(gflat, src2, dst2, n_pad, dh, column_split, mul, off):
    e_chunks, ch = src2.shape
    assert ch == _CH
    nc, ns = 2, 16
    n_ch_w = e_chunks // ns if column_split else e_chunks // (ns * nc)
    nsup = _NSUP if n_ch_w % _NSUP == 0 else 8
    n_sup = n_ch_w // nsup
    nbuf = min(nsup, 4 if dh > 16 else 8)
    rows_w = n_pad // ns
    nz = rows_w // _ZR
    assert n_ch_w % 8 == 0 and n_sup * nsup == n_ch_w
    assert rows_w % 8 == 0 and nz * _ZR == rows_w
    mesh = plsc.VectorSubcoreMesh(core_axis_name="c", subcore_axis_name="s")

    scratch = [
        pltpu.VMEM((nsup, _CH), jnp.int32),
        pltpu.VMEM((nsup, _CH), jnp.int32),
        pltpu.VMEM((nsup, _CH), jnp.int32),
        pltpu.VMEM((nbuf, _CH, dh), _F32),
        pltpu.VMEM((_ZR, dh), _F32),
        pltpu.VMEM_SHARED((n_pad, dh), _F32),
        [pltpu.SemaphoreType.DMA] * nbuf,
        [pltpu.SemaphoreType.DMA] * nbuf,
    ]

    @functools.partial(pl.kernel,
                       out_type=jax.ShapeDtypeStruct((n_pad, nc * dh), _F32),
                       mesh=mesh, scratch_types=scratch,
                       compiler_params=pltpu.CompilerParams(
                           use_tc_tiling_on_sc=False))
    def k(g_hbm, src_hbm, dst_hbm, out_hbm, sraw, sadj, dstv, rowb, zb, acc,
          gsems, ssems):
        c = lax.axis_index("c")
        s = lax.axis_index("s")

        def _zb(i, carry):
            for t in range(dh // 16):
                zb[i, pl.ds(t * 16, 16)] = jnp.zeros((16,), _F32)
            return carry
        lax.fori_loop(0, _ZR, _zb, 0)

        r0 = s * rows_w

        def _za(i, carry):
            pltpu.sync_copy(zb, acc.at[pl.ds(r0 + i * _ZR, _ZR)])
            return carry
        lax.fori_loop(0, nz, _za, 0)
        plsc.subcore_barrier()

        base_ch = s * n_ch_w if column_split else (s * nc + c) * n_ch_w

        def _sup(k0, carry):
            row = base_ch + k0 * nsup
            pltpu.sync_copy(src_hbm.at[pl.ds(row, nsup)], sraw)
            pltpu.sync_copy(dst_hbm.at[pl.ds(row, nsup)], dstv)
            adj = off + (c if column_split else 0)
            def _adj(i, cc):
                for t in range(_CH // 16):
                    v = sraw[i, pl.ds(t * 16, 16)]
                    sadj[i, pl.ds(t * 16, 16)] = v * mul + adj
                return cc
            lax.fori_loop(0, nsup, _adj, 0)
            idxs = sadj

            gds = {b: pltpu.async_copy(g_hbm.at[idxs.at[b]], rowb.at[b],
                                       gsems[b]) for b in range(nbuf)}
            sds = {}
            for b in range(nsup):
                pb = b - 1
                if pb >= 0 and pb + nbuf < nsup:
                    sds[pb].wait()
                    pbb = pb % nbuf
                    gds[pb + nbuf] = pltpu.async_copy(
                        g_hbm.at[idxs.at[pb + nbuf]], rowb.at[pbb],
                        gsems[pbb])
                gds[b].wait()
                sds[b] = pltpu.async_copy(rowb.at[b % nbuf],
                                          acc.at[dstv.at[b]],
                                          ssems[b % nbuf], add=True)
            for b in range(max(0, nsup - nbuf), nsup):
                sds[b].wait()
            return carry
        lax.fori_loop(0, n_sup, _sup, 0)
        plsc.subcore_barrier()

        def _wb(i, carry):
            pltpu.sync_copy(acc.at[pl.ds(r0 + i * _ZR, _ZR)],
                            out_hbm.at[pl.ds(r0 + i * _ZR, _ZR),
                                       pl.ds(c * dh, dh)])
            return carry
        lax.fori_loop(0, nz, _wb, 0)

    return k(gflat, src2, dst2)


def _ceil_to(v, m):
    return ((v + m - 1) // m) * m


def kernel(x, edge_index, params):
    n = x.shape[0]
    e = edge_index.shape[1]
    src = edge_index[0]
    dst = edge_index[1]

    prm = dict(params)
    for k in list(prm):
        if k.endswith("_b"):
            prm[k] = prm[k].reshape(1, -1)

    ep = _ceil_to(e, 32 * _CH * 8)
    n_pad = _ceil_to(n + 1, 16 * _ZR)
    p = ep - e
    pad_ar = jnp.arange(p, dtype=jnp.int32)
    srcp = jnp.concatenate([src, pad_ar & 16383]).reshape(ep // _CH, _CH)
    dstp = jnp.concatenate([dst, n + (pad_ar & 127)]).reshape(ep // _CH, _CH)

    p1 = _pass_a(x, prm, n)
    agg1 = _sc_segsum(x.reshape(2 * n, 32), srcp, dstp, n_pad, 32, True,
                      2, 0)
    m2 = _pass_bc(p1, agg1, prm, n, 2)
    agg2 = _sc_segsum(m2.reshape(4 * n, 16), srcp, dstp, n_pad, 16, True,
                      4, 2)
    m3 = _pass_bc(m2, agg2, prm, n, 3)
    agg3 = _sc_segsum(m3.reshape(2 * n, 16), srcp, dstp, n_pad, 16, False,
                      2, 1)
    return _pass_d(m3, agg3, prm, n)
